# baseline (speedup 1.0000x reference)
import jax
import jax.numpy as jnp
from jax.experimental import pallas as pl

N = 4096
R = 4
INDIM = 128
OUTDIM = 16

BM = 256


def _xw_kernel(x_ref, w_ref, xw_ref):
    xw_ref[0] = jnp.dot(x_ref[...], w_ref[0],
                        preferred_element_type=jnp.float32)


def _agg_kernel(a_ref, xw_ref, o_ref):
    acc = jnp.dot(a_ref[...], xw_ref[...],
                  preferred_element_type=jnp.float32)
    o_ref[...] = jnp.maximum(acc, 0.0)


def kernel(X, A, W):
    Wv = W.reshape(R, INDIM, OUTDIM)

    xw = pl.pallas_call(
        _xw_kernel,
        grid=(R,),
        in_specs=[
            pl.BlockSpec((N, INDIM), lambda r: (0, 0)),
            pl.BlockSpec((1, INDIM, OUTDIM), lambda r: (r, 0, 0)),
        ],
        out_specs=pl.BlockSpec((1, N, OUTDIM), lambda r: (r, 0, 0)),
        out_shape=jax.ShapeDtypeStruct((R, N, OUTDIM), jnp.float32),
    )(X, Wv)
    xw = xw.reshape(R * N, OUTDIM)

    out = pl.pallas_call(
        _agg_kernel,
        grid=(N // BM,),
        in_specs=[
            pl.BlockSpec((BM, R * N), lambda m: (m, 0)),
            pl.BlockSpec((R * N, OUTDIM), lambda m: (0, 0)),
        ],
        out_specs=pl.BlockSpec((BM, OUTDIM), lambda m: (m, 0)),
        out_shape=jax.ShapeDtypeStruct((N, OUTDIM), jnp.float32),
    )(A, xw)
    return out


# speedup vs baseline: 1.0931x; 1.0931x over previous
import jax
import jax.numpy as jnp
from jax.experimental import pallas as pl
from jax.experimental.pallas import tpu as pltpu

N = 4096
R = 4
INDIM = 128
OUTDIM = 16

BM = 256


def _mrgcn_kernel(x_ref, w_ref, a_ref, o_ref, xw_ref):
    @pl.when(pl.program_id(0) == 0)
    def _():
        x = x_ref[...]
        for r in range(R):
            xw_ref[r * N:(r + 1) * N, :] = jnp.dot(
                x, w_ref[r], preferred_element_type=jnp.float32)

    acc = jnp.dot(a_ref[...], xw_ref[...],
                  preferred_element_type=jnp.float32)
    o_ref[...] = jnp.maximum(acc, 0.0)


def kernel(X, A, W):
    Wv = W.reshape(R, INDIM, OUTDIM)
    return pl.pallas_call(
        _mrgcn_kernel,
        grid=(N // BM,),
        in_specs=[
            pl.BlockSpec((N, INDIM), lambda m: (0, 0)),
            pl.BlockSpec((R, INDIM, OUTDIM), lambda m: (0, 0, 0)),
            pl.BlockSpec((BM, R * N), lambda m: (m, 0)),
        ],
        out_specs=pl.BlockSpec((BM, OUTDIM), lambda m: (m, 0)),
        out_shape=jax.ShapeDtypeStruct((N, OUTDIM), jnp.float32),
        scratch_shapes=[pltpu.VMEM((R * N, OUTDIM), jnp.float32)],
    )(X, Wv, A)
